# baseline (speedup 1.0000x reference)
import functools

import jax
import jax.numpy as jnp
from jax import lax
from jax.experimental import pallas as pl
from jax.experimental.pallas import tpu as pltpu
from jax.experimental.pallas import tpu_sc as plsc

NC, NS = 2, 16
NW = NC * NS
CH = 128


def _mesh():
    return plsc.VectorSubcoreMesh(
        core_axis_name="c", subcore_axis_name="s",
        num_cores=NC, num_subcores=NS)


def _sc_scatter_add(edges, dst, n_nodes):
    E, D = edges.shape
    n_chunks = E // CH
    assert n_chunks * CH == E
    z_rows = 64
    rows_per_tile = -(-n_nodes // (NS * z_rows)) * z_rows
    n_pad = rows_per_tile * NS

    @functools.partial(
        pl.kernel,
        out_type=jax.ShapeDtypeStruct((NC, n_pad, D), jnp.float32),
        mesh=_mesh(),
        scratch_types=[
            pltpu.VMEM((2, CH), jnp.int32),
            pltpu.VMEM((2, CH, D), jnp.float32),
            pltpu.VMEM((z_rows, D), jnp.float32),
            pltpu.VMEM_SHARED((n_pad, D), jnp.float32),
            pltpu.SemaphoreType.DMA((2,)),
            pltpu.SemaphoreType.DMA((2,)),
        ],
    )
    def body(edges_hbm, dst_hbm, out_hbm, idx_v, rows_v, zero_v, acc_sh,
             sem_i, sem_r):
        c = lax.axis_index("c")
        s = lax.axis_index("s")
        w = s * NC + c

        def zf(i, carry):
            for j in range(D // 16):
                zero_v[i, pl.ds(16 * j, 16)] = jnp.zeros((16,), jnp.float32)
            return carry
        lax.fori_loop(0, z_rows, zf, 0)
        for j in range(rows_per_tile // z_rows):
            pltpu.sync_copy(
                zero_v,
                acc_sh.at[pl.ds(s * rows_per_tile + j * z_rows, z_rows)])
        plsc.subcore_barrier()

        cnt = n_chunks // NW + jnp.where(w < (n_chunks % NW), 1, 0)

        def base_of(t):
            return (w + NW * t) * CH

        def start_load(t, b):
            pltpu.async_copy(dst_hbm.at[pl.ds(base_of(t), CH)],
                             idx_v.at[b], sem_i.at[b])
            pltpu.async_copy(edges_hbm.at[pl.ds(base_of(t), CH)],
                             rows_v.at[b], sem_r.at[b])

        def wait_load(t, b):
            pltpu.make_async_copy(dst_hbm.at[pl.ds(base_of(t), CH)],
                                  idx_v.at[b], sem_i.at[b]).wait()
            pltpu.make_async_copy(edges_hbm.at[pl.ds(base_of(t), CH)],
                                  rows_v.at[b], sem_r.at[b]).wait()

        start_load(0, 0)

        def step_b(t, b):
            wait_load(t, b)

            @pl.when(t + 1 < cnt)
            def _():
                start_load(t + 1, 1 - b)

            pltpu.sync_copy(rows_v.at[b], acc_sh.at[idx_v.at[b]], add=True)

        def step(t, carry):
            @pl.when(t % 2 == 0)
            def _():
                step_b(t, 0)

            @pl.when(t % 2 == 1)
            def _():
                step_b(t, 1)
            return carry
        lax.fori_loop(0, cnt, step, 0)
        plsc.subcore_barrier()

        pltpu.sync_copy(
            acc_sh.at[pl.ds(s * rows_per_tile, rows_per_tile)],
            out_hbm.at[c, pl.ds(s * rows_per_tile, rows_per_tile)])

    return body(edges, dst)


def _sc_gather_sum(a_tab, b_tab, src, dst):
    _, D = a_tab.shape
    E = src.shape[0]
    n_chunks = E // CH

    @functools.partial(
        pl.kernel,
        out_type=jax.ShapeDtypeStruct((E, D), jnp.float32),
        mesh=_mesh(),
        scratch_types=[
            pltpu.VMEM((2, CH), jnp.int32),
            pltpu.VMEM((2, CH), jnp.int32),
            pltpu.VMEM((2, CH, D), jnp.float32),
            pltpu.VMEM((2, CH, D), jnp.float32),
            pltpu.SemaphoreType.DMA((2,)),
            pltpu.SemaphoreType.DMA((2,)),
            pltpu.SemaphoreType.DMA((2,)),
            pltpu.SemaphoreType.DMA((2,)),
        ],
    )
    def body(a_hbm, b_hbm, src_hbm, dst_hbm, out_hbm,
             si_v, di_v, buf_a, buf_b, sem_i, sem_j, sem_g, sem_o):
        c = lax.axis_index("c")
        s = lax.axis_index("s")
        w = s * NC + c

        cnt = n_chunks // NW + jnp.where(w < (n_chunks % NW), 1, 0)

        def base_of(t):
            return (w + NW * t) * CH

        def start_idx(t, b):
            pltpu.async_copy(src_hbm.at[pl.ds(base_of(t), CH)],
                             si_v.at[b], sem_i.at[b])
            pltpu.async_copy(dst_hbm.at[pl.ds(base_of(t), CH)],
                             di_v.at[b], sem_j.at[b])

        def wait_idx(t, b):
            pltpu.make_async_copy(src_hbm.at[pl.ds(base_of(t), CH)],
                                  si_v.at[b], sem_i.at[b]).wait()
            pltpu.make_async_copy(dst_hbm.at[pl.ds(base_of(t), CH)],
                                  di_v.at[b], sem_j.at[b]).wait()

        def start_gathers(b):
            pltpu.async_copy(a_hbm.at[si_v.at[b]], buf_a.at[b], sem_g.at[b])
            pltpu.async_copy(b_hbm.at[di_v.at[b]], buf_b.at[b], sem_g.at[b])

        def wait_gathers(b):
            pltpu.make_async_copy(a_hbm.at[si_v.at[b]], buf_a.at[b],
                                  sem_g.at[b]).wait()
            pltpu.make_async_copy(b_hbm.at[di_v.at[b]], buf_b.at[b],
                                  sem_g.at[b]).wait()

        def start_out(t, b):
            pltpu.async_copy(buf_a.at[b], out_hbm.at[pl.ds(base_of(t), CH)],
                             sem_o.at[b])

        def wait_out(t, b):
            pltpu.make_async_copy(buf_a.at[b],
                                  out_hbm.at[pl.ds(base_of(t), CH)],
                                  sem_o.at[b]).wait()

        def add_and_out(t, b):
            def add_row(i, cc):
                for j in range(D // 16):
                    sl = pl.ds(16 * j, 16)
                    buf_a[b, i, sl] = buf_a[b, i, sl] + buf_b[b, i, sl]
                return cc
            lax.fori_loop(0, CH, add_row, 0)
            start_out(t, b)

        start_idx(0, 0)

        def step_b(t, b):
            wait_idx(t, b)

            @pl.when(t >= 2)
            def _():
                wait_out(t - 2, b)
            start_gathers(b)

            @pl.when(t >= 1)
            def _():
                wait_gathers(1 - b)

                @pl.when(t + 1 < cnt)
                def _():
                    start_idx(t + 1, 1 - b)
                add_and_out(t - 1, 1 - b)

            @pl.when((t == 0) & (cnt > 1))
            def _():
                start_idx(1, 1)

        def step(t, carry):
            @pl.when(t % 2 == 0)
            def _():
                step_b(t, 0)

            @pl.when(t % 2 == 1)
            def _():
                step_b(t, 1)
            return carry
        lax.fori_loop(0, cnt, step, 0)

        def tail_b(bl):
            @pl.when(cnt >= 2)
            def _():
                wait_out(cnt - 2, 1 - bl)
            wait_gathers(bl)
            add_and_out(cnt - 1, bl)
            wait_out(cnt - 1, bl)

        @pl.when((cnt - 1) % 2 == 0)
        def _():
            tail_b(0)

        @pl.when((cnt - 1) % 2 == 1)
        def _():
            tail_b(1)

    return body(a_tab, b_tab, src, dst)


def _ln(h, g, b, eps=1e-5):
    m = jnp.mean(h, axis=-1, keepdims=True)
    v = jnp.mean((h - m) ** 2, axis=-1, keepdims=True)
    return (h - m) * lax.rsqrt(v + eps) * g + b


def _tc_node(nodes, msg_parts, nW0x, nW0m, nb0, ng0, nbe0,
             nW1, nb1, ng1, nbe1, eW0a, eW0b):
    N, D = nodes.shape
    BN = 10000
    assert N % BN == 0
    grid = (N // BN,)

    def body(x_ref, m_ref, w0x_ref, w0m_ref, b0_ref, g0_ref, be0_ref,
             w1_ref, b1_ref, g1_ref, be1_ref, wa_ref, wb_ref,
             nn_ref, a_ref, b_ref):
        x = x_ref[...]
        m = m_ref[0] + m_ref[1]
        h = (jnp.dot(x, w0x_ref[...], preferred_element_type=jnp.float32)
             + jnp.dot(m, w0m_ref[...], preferred_element_type=jnp.float32)
             + b0_ref[...])
        h = jnp.maximum(_ln(h, g0_ref[...], be0_ref[...]), 0.0)
        h = jnp.dot(h, w1_ref[...], preferred_element_type=jnp.float32) + b1_ref[...]
        h = jnp.maximum(_ln(h, g1_ref[...], be1_ref[...]), 0.0)
        nn = h + x
        nn_ref[...] = nn
        a_ref[...] = jnp.dot(nn, wa_ref[...], preferred_element_type=jnp.float32)
        b_ref[...] = jnp.dot(nn, wb_ref[...], preferred_element_type=jnp.float32)

    blk = pl.BlockSpec((BN, D), lambda i: (i, 0))
    wspec = pl.BlockSpec((D, D), lambda i: (0, 0))
    vspec = pl.BlockSpec((1, D), lambda i: (0, 0))
    return pl.pallas_call(
        body,
        grid=grid,
        in_specs=[blk, pl.BlockSpec((2, BN, D), lambda i: (0, i, 0)),
                  wspec, wspec, vspec, vspec, vspec,
                  wspec, vspec, vspec, vspec, wspec, wspec],
        out_specs=[blk, blk, blk],
        out_shape=[jax.ShapeDtypeStruct((N, D), jnp.float32)] * 3,
    )(nodes, msg_parts, nW0x, nW0m, nb0, ng0, nbe0,
      nW1, nb1, ng1, nbe1, eW0a, eW0b)


def _tc_edge(edges, gsum, eW0e, eb0, eg0, ebe0, eW1, eb1, eg1, ebe1):
    E, D = edges.shape
    BE = 8000
    assert E % BE == 0
    grid = (E // BE,)

    def body(x_ref, g_ref, w0_ref, b0_ref, g0_ref, be0_ref,
             w1_ref, b1_ref, g1_ref, be1_ref, out_ref):
        x = x_ref[...]
        h = (jnp.dot(x.astype(jnp.bfloat16), w0_ref[...],
                     preferred_element_type=jnp.float32)
             + g_ref[...] + b0_ref[...])
        h = jnp.maximum(_ln(h, g0_ref[...], be0_ref[...]), 0.0)
        h = jnp.dot(h.astype(jnp.bfloat16), w1_ref[...],
                    preferred_element_type=jnp.float32) + b1_ref[...]
        h = jnp.tanh(_ln(h, g1_ref[...], be1_ref[...]))
        out_ref[...] = h + x

    blk = pl.BlockSpec((BE, D), lambda i: (i, 0))
    wspec = pl.BlockSpec((D, D), lambda i: (0, 0))
    vspec = pl.BlockSpec((1, D), lambda i: (0, 0))
    return pl.pallas_call(
        body,
        grid=grid,
        in_specs=[blk, blk, wspec, vspec, vspec, vspec,
                  wspec, vspec, vspec, vspec],
        out_specs=blk,
        out_shape=jax.ShapeDtypeStruct((E, D), jnp.float32),
    )(edges, gsum, eW0e.astype(jnp.bfloat16), eb0, eg0, ebe0,
      eW1.astype(jnp.bfloat16), eb1, eg1, ebe1)


def kernel(nodes, edges, graph,
           eW0, eb0, eg0, ebe0, eW1, eb1, eg1, ebe1,
           nW0, nb0, ng0, nbe0, nW1, nb1, ng1, nbe1):
    N, D = nodes.shape
    E_sz = edges.shape[0]
    src = graph[0]
    dst = graph[1]

    msg_parts = _sc_scatter_add(edges, dst, N)

    r = lambda v: v.reshape(1, D)
    nodes_new, a_tab, b_tab = _tc_node(
        nodes, msg_parts,
        nW0[:D], nW0[D:], r(nb0), r(ng0), r(nbe0),
        nW1, r(nb1), r(ng1), r(nbe1),
        eW0[:D], eW0[D:2 * D])

    gsum = _sc_gather_sum(a_tab, b_tab, src, dst)

    edges_new = _tc_edge(
        edges, gsum, eW0[2 * D:], r(eb0), r(eg0), r(ebe0),
        eW1, r(eb1), r(eg1), r(ebe1))
    return (nodes_new, edges_new)


# speedup vs baseline: 1.0164x; 1.0164x over previous
import functools

import jax
import jax.numpy as jnp
from jax import lax
from jax.experimental import pallas as pl
from jax.experimental.pallas import tpu as pltpu
from jax.experimental.pallas import tpu_sc as plsc

NC, NS = 2, 16
NW = NC * NS
CH = 128


def _mesh():
    return plsc.VectorSubcoreMesh(
        core_axis_name="c", subcore_axis_name="s",
        num_cores=NC, num_subcores=NS)


def _sc_scatter_add(edges, dst, n_nodes):
    E, D = edges.shape
    n_chunks = E // CH
    assert n_chunks * CH == E
    z_rows = 64
    rows_per_tile = -(-n_nodes // (NS * z_rows)) * z_rows
    n_pad = rows_per_tile * NS

    @functools.partial(
        pl.kernel,
        out_type=jax.ShapeDtypeStruct((NC, n_pad, D), jnp.float32),
        mesh=_mesh(),
        scratch_types=[
            pltpu.VMEM((2, CH), jnp.int32),
            pltpu.VMEM((2, CH, D), jnp.float32),
            pltpu.VMEM((z_rows, D), jnp.float32),
            pltpu.VMEM_SHARED((n_pad, D), jnp.float32),
            pltpu.SemaphoreType.DMA((2,)),
            pltpu.SemaphoreType.DMA((2,)),
        ],
    )
    def body(edges_hbm, dst_hbm, out_hbm, idx_v, rows_v, zero_v, acc_sh,
             sem_i, sem_r):
        c = lax.axis_index("c")
        s = lax.axis_index("s")
        w = s * NC + c

        def zf(i, carry):
            for j in range(D // 16):
                zero_v[i, pl.ds(16 * j, 16)] = jnp.zeros((16,), jnp.float32)
            return carry
        lax.fori_loop(0, z_rows, zf, 0)
        for j in range(rows_per_tile // z_rows):
            pltpu.sync_copy(
                zero_v,
                acc_sh.at[pl.ds(s * rows_per_tile + j * z_rows, z_rows)])
        plsc.subcore_barrier()

        cnt = n_chunks // NW + jnp.where(w < (n_chunks % NW), 1, 0)

        def base_of(t):
            return (w + NW * t) * CH

        def start_load(t, b):
            pltpu.async_copy(dst_hbm.at[pl.ds(base_of(t), CH)],
                             idx_v.at[b], sem_i.at[b])
            pltpu.async_copy(edges_hbm.at[pl.ds(base_of(t), CH)],
                             rows_v.at[b], sem_r.at[b])

        def wait_load(t, b):
            pltpu.make_async_copy(dst_hbm.at[pl.ds(base_of(t), CH)],
                                  idx_v.at[b], sem_i.at[b]).wait()
            pltpu.make_async_copy(edges_hbm.at[pl.ds(base_of(t), CH)],
                                  rows_v.at[b], sem_r.at[b]).wait()

        start_load(0, 0)

        def step_b(t, b):
            wait_load(t, b)

            @pl.when(t + 1 < cnt)
            def _():
                start_load(t + 1, 1 - b)

            pltpu.sync_copy(rows_v.at[b], acc_sh.at[idx_v.at[b]], add=True)

        def step(t, carry):
            @pl.when(t % 2 == 0)
            def _():
                step_b(t, 0)

            @pl.when(t % 2 == 1)
            def _():
                step_b(t, 1)
            return carry
        lax.fori_loop(0, cnt, step, 0)
        plsc.subcore_barrier()

        pltpu.sync_copy(
            acc_sh.at[pl.ds(s * rows_per_tile, rows_per_tile)],
            out_hbm.at[c, pl.ds(s * rows_per_tile, rows_per_tile)])

    return body(edges, dst)


def _sc_gather_sum(a_tab, b_tab, src, dst):
    _, D = a_tab.shape
    E = src.shape[0]
    n_chunks = E // CH
    NB = 3

    @functools.partial(
        pl.kernel,
        out_type=jax.ShapeDtypeStruct((E, D), jnp.float32),
        mesh=_mesh(),
        scratch_types=[
            pltpu.VMEM((NB, CH), jnp.int32),
            pltpu.VMEM((NB, CH), jnp.int32),
            pltpu.VMEM((NB, CH, D), jnp.float32),
            pltpu.VMEM((NB, CH, D), jnp.float32),
            pltpu.SemaphoreType.DMA((NB,)),
            pltpu.SemaphoreType.DMA((NB,)),
            pltpu.SemaphoreType.DMA((NB,)),
            pltpu.SemaphoreType.DMA((NB,)),
        ],
    )
    def body(a_hbm, b_hbm, src_hbm, dst_hbm, out_hbm,
             si_v, di_v, buf_a, buf_b, sem_i, sem_j, sem_g, sem_o):
        c = lax.axis_index("c")
        s = lax.axis_index("s")
        w = s * NC + c

        cnt = n_chunks // NW + jnp.where(w < (n_chunks % NW), 1, 0)

        def base_of(t):
            return (w + NW * t) * CH

        def start_idx(t, b):
            pltpu.async_copy(src_hbm.at[pl.ds(base_of(t), CH)],
                             si_v.at[b], sem_i.at[b])
            pltpu.async_copy(dst_hbm.at[pl.ds(base_of(t), CH)],
                             di_v.at[b], sem_j.at[b])

        def wait_idx(t, b):
            pltpu.make_async_copy(src_hbm.at[pl.ds(base_of(t), CH)],
                                  si_v.at[b], sem_i.at[b]).wait()
            pltpu.make_async_copy(dst_hbm.at[pl.ds(base_of(t), CH)],
                                  di_v.at[b], sem_j.at[b]).wait()

        def start_gathers(b):
            pltpu.async_copy(a_hbm.at[si_v.at[b]], buf_a.at[b], sem_g.at[b])
            pltpu.async_copy(b_hbm.at[di_v.at[b]], buf_b.at[b], sem_g.at[b])

        def wait_gathers(b):
            pltpu.make_async_copy(a_hbm.at[si_v.at[b]], buf_a.at[b],
                                  sem_g.at[b]).wait()
            pltpu.make_async_copy(b_hbm.at[di_v.at[b]], buf_b.at[b],
                                  sem_g.at[b]).wait()

        def start_out(t, b):
            pltpu.async_copy(buf_a.at[b], out_hbm.at[pl.ds(base_of(t), CH)],
                             sem_o.at[b])

        def wait_out(t, b):
            pltpu.make_async_copy(buf_a.at[b],
                                  out_hbm.at[pl.ds(base_of(t), CH)],
                                  sem_o.at[b]).wait()

        def add_and_out(t, b):
            def add_row(i, cc):
                for j in range(D // 16):
                    sl = pl.ds(16 * j, 16)
                    buf_a[b, i, sl] = buf_a[b, i, sl] + buf_b[b, i, sl]
                return cc
            lax.fori_loop(0, CH, add_row, 0)
            start_out(t, b)

        start_idx(0, 0)

        def step_b(t, b):
            wait_idx(t, b)

            @pl.when(t >= NB)
            def _():
                wait_out(t - NB, b)
            start_gathers(b)

            @pl.when(t + 1 < cnt)
            def _():
                start_idx(t + 1, (b + 1) % NB)

            @pl.when(t >= 1)
            def _():
                wait_gathers((b + NB - 1) % NB)
                add_and_out(t - 1, (b + NB - 1) % NB)

        def step(t, carry):
            for bb in range(NB):
                @pl.when(t % NB == bb)
                def _(bb=bb):
                    step_b(t, bb)
            return carry
        lax.fori_loop(0, cnt, step, 0)

        def tail_b(bl):
            wait_gathers(bl)
            add_and_out(cnt - 1, bl)
            wait_out(cnt - 3, (bl + 1) % NB)
            wait_out(cnt - 2, (bl + 2) % NB)
            wait_out(cnt - 1, bl)

        for bb in range(NB):
            @pl.when((cnt - 1) % NB == bb)
            def _(bb=bb):
                tail_b(bb)

    return body(a_tab, b_tab, src, dst)


def _ln(h, g, b, eps=1e-5):
    m = jnp.mean(h, axis=-1, keepdims=True)
    v = jnp.mean((h - m) ** 2, axis=-1, keepdims=True)
    return (h - m) * lax.rsqrt(v + eps) * g + b


def _tc_node(nodes, msg_parts, nW0x, nW0m, nb0, ng0, nbe0,
             nW1, nb1, ng1, nbe1, eW0a, eW0b):
    N, D = nodes.shape
    BN = 2000
    assert N % BN == 0
    grid = (N // BN,)

    def body(x_ref, m_ref, w0x_ref, w0m_ref, b0_ref, g0_ref, be0_ref,
             w1_ref, b1_ref, g1_ref, be1_ref, wa_ref, wb_ref,
             nn_ref, a_ref, b_ref):
        x = x_ref[...]
        m = m_ref[0] + m_ref[1]
        h = (jnp.dot(x, w0x_ref[...], preferred_element_type=jnp.float32)
             + jnp.dot(m, w0m_ref[...], preferred_element_type=jnp.float32)
             + b0_ref[...])
        h = jnp.maximum(_ln(h, g0_ref[...], be0_ref[...]), 0.0)
        h = jnp.dot(h, w1_ref[...], preferred_element_type=jnp.float32) + b1_ref[...]
        h = jnp.maximum(_ln(h, g1_ref[...], be1_ref[...]), 0.0)
        nn = h + x
        nn_ref[...] = nn
        a_ref[...] = jnp.dot(nn, wa_ref[...], preferred_element_type=jnp.float32)
        b_ref[...] = jnp.dot(nn, wb_ref[...], preferred_element_type=jnp.float32)

    blk = pl.BlockSpec((BN, D), lambda i: (i, 0))
    wspec = pl.BlockSpec((D, D), lambda i: (0, 0))
    vspec = pl.BlockSpec((1, D), lambda i: (0, 0))
    return pl.pallas_call(
        body,
        grid=grid,
        in_specs=[blk, pl.BlockSpec((2, BN, D), lambda i: (0, i, 0)),
                  wspec, wspec, vspec, vspec, vspec,
                  wspec, vspec, vspec, vspec, wspec, wspec],
        out_specs=[blk, blk, blk],
        out_shape=[jax.ShapeDtypeStruct((N, D), jnp.float32)] * 3,
    )(nodes, msg_parts, nW0x, nW0m, nb0, ng0, nbe0,
      nW1, nb1, ng1, nbe1, eW0a, eW0b)


def _tc_edge(edges, gsum, eW0e, eb0, eg0, ebe0, eW1, eb1, eg1, ebe1):
    E, D = edges.shape
    BE = 8000
    assert E % BE == 0
    grid = (E // BE,)

    def body(x_ref, g_ref, w0_ref, b0_ref, g0_ref, be0_ref,
             w1_ref, b1_ref, g1_ref, be1_ref, out_ref):
        x = x_ref[...]
        h = (jnp.dot(x.astype(jnp.bfloat16), w0_ref[...],
                     preferred_element_type=jnp.float32)
             + g_ref[...] + b0_ref[...])
        h = jnp.maximum(_ln(h, g0_ref[...], be0_ref[...]), 0.0)
        h = jnp.dot(h.astype(jnp.bfloat16), w1_ref[...],
                    preferred_element_type=jnp.float32) + b1_ref[...]
        h = jnp.tanh(_ln(h, g1_ref[...], be1_ref[...]))
        out_ref[...] = h + x

    blk = pl.BlockSpec((BE, D), lambda i: (i, 0))
    wspec = pl.BlockSpec((D, D), lambda i: (0, 0))
    vspec = pl.BlockSpec((1, D), lambda i: (0, 0))
    return pl.pallas_call(
        body,
        grid=grid,
        in_specs=[blk, blk, wspec, vspec, vspec, vspec,
                  wspec, vspec, vspec, vspec],
        out_specs=blk,
        out_shape=jax.ShapeDtypeStruct((E, D), jnp.float32),
    )(edges, gsum, eW0e.astype(jnp.bfloat16), eb0, eg0, ebe0,
      eW1.astype(jnp.bfloat16), eb1, eg1, ebe1)


def kernel(nodes, edges, graph,
           eW0, eb0, eg0, ebe0, eW1, eb1, eg1, ebe1,
           nW0, nb0, ng0, nbe0, nW1, nb1, ng1, nbe1):
    N, D = nodes.shape
    E_sz = edges.shape[0]
    src = graph[0]
    dst = graph[1]

    msg_parts = _sc_scatter_add(edges, dst, N)

    r = lambda v: v.reshape(1, D)
    nodes_new, a_tab, b_tab = _tc_node(
        nodes, msg_parts,
        nW0[:D], nW0[D:], r(nb0), r(ng0), r(nbe0),
        nW1, r(nb1), r(ng1), r(nbe1),
        eW0[:D], eW0[D:2 * D])

    gsum = _sc_gather_sum(a_tab, b_tab, src, dst)

    edges_new = _tc_edge(
        edges, gsum, eW0[2 * D:], r(eb0), r(eg0), r(ebe0),
        eW1, r(eb1), r(eg1), r(ebe1))
    return (nodes_new, edges_new)


# speedup vs baseline: 1.0749x; 1.0576x over previous
import functools

import jax
import jax.numpy as jnp
from jax import lax
from jax.experimental import pallas as pl
from jax.experimental.pallas import tpu as pltpu
from jax.experimental.pallas import tpu_sc as plsc

NC, NS = 2, 16
NW = NC * NS
CH = 128


def _mesh():
    return plsc.VectorSubcoreMesh(
        core_axis_name="c", subcore_axis_name="s",
        num_cores=NC, num_subcores=NS)


def _sc_scatter_add(edges, dst, n_nodes):
    E, D = edges.shape
    n_chunks = E // CH
    assert n_chunks * CH == E
    rows_per_tile = -(-n_nodes // (NS * 8)) * 8
    n_pad = rows_per_tile * NS
    NB = 3

    @functools.partial(
        pl.kernel,
        out_type=jax.ShapeDtypeStruct((NC, n_pad, D), jnp.float32),
        mesh=_mesh(),
        scratch_types=[
            pltpu.VMEM((NB, CH), jnp.int32),
            pltpu.VMEM((NB, CH, D), jnp.float32),
            pltpu.VMEM_SHARED((n_pad, D), jnp.float32),
            pltpu.SemaphoreType.DMA((NB,)),
            pltpu.SemaphoreType.DMA((NB,)),
        ],
    )
    def body(edges_hbm, dst_hbm, out_hbm, idx_v, rows_v, acc_sh,
             sem_i, sem_r):
        c = lax.axis_index("c")
        s = lax.axis_index("s")
        w = s * NC + c

        def zf(i, carry):
            for j in range(D // 16):
                rows_v[0, i, pl.ds(16 * j, 16)] = jnp.zeros((16,), jnp.float32)
            return carry
        lax.fori_loop(0, CH, zf, 0)
        for j in range(rows_per_tile // CH):
            pltpu.sync_copy(
                rows_v.at[0],
                acc_sh.at[pl.ds(s * rows_per_tile + j * CH, CH)])
        rem = rows_per_tile % CH
        if rem:
            pltpu.sync_copy(
                rows_v.at[0, pl.ds(0, rem)],
                acc_sh.at[pl.ds(
                    s * rows_per_tile + (rows_per_tile // CH) * CH, rem)])
        plsc.subcore_barrier()

        cnt = n_chunks // NW + jnp.where(w < (n_chunks % NW), 1, 0)

        def base_of(t):
            return (w + NW * t) * CH

        def start_load(t, b):
            pltpu.async_copy(dst_hbm.at[pl.ds(base_of(t), CH)],
                             idx_v.at[b], sem_i.at[b])
            pltpu.async_copy(edges_hbm.at[pl.ds(base_of(t), CH)],
                             rows_v.at[b], sem_r.at[b])

        def wait_load(t, b):
            pltpu.make_async_copy(dst_hbm.at[pl.ds(base_of(t), CH)],
                                  idx_v.at[b], sem_i.at[b]).wait()
            pltpu.make_async_copy(edges_hbm.at[pl.ds(base_of(t), CH)],
                                  rows_v.at[b], sem_r.at[b]).wait()

        start_load(0, 0)

        @pl.when(cnt > 1)
        def _():
            start_load(1, 1)

        def step_b(t, b):
            wait_load(t, b)

            @pl.when(t + 2 < cnt)
            def _():
                start_load(t + 2, (b + 2) % NB)

            pltpu.sync_copy(rows_v.at[b], acc_sh.at[idx_v.at[b]], add=True)

        def step(t, carry):
            for bb in range(NB):
                @pl.when(t % NB == bb)
                def _(bb=bb):
                    step_b(t, bb)
            return carry
        lax.fori_loop(0, cnt, step, 0)
        plsc.subcore_barrier()

        pltpu.sync_copy(
            acc_sh.at[pl.ds(s * rows_per_tile, rows_per_tile)],
            out_hbm.at[c, pl.ds(s * rows_per_tile, rows_per_tile)])

    return body(edges, dst)


def _sc_gather_sum(a_tab, b_tab, src, dst):
    _, D = a_tab.shape
    E = src.shape[0]
    n_chunks = E // CH
    NB = 3

    @functools.partial(
        pl.kernel,
        out_type=jax.ShapeDtypeStruct((E, D), jnp.float32),
        mesh=_mesh(),
        scratch_types=[
            pltpu.VMEM((NB, CH), jnp.int32),
            pltpu.VMEM((NB, CH), jnp.int32),
            pltpu.VMEM((NB, CH, D), jnp.float32),
            pltpu.VMEM((NB, CH, D), jnp.float32),
            pltpu.SemaphoreType.DMA((NB,)),
            pltpu.SemaphoreType.DMA((NB,)),
            pltpu.SemaphoreType.DMA((NB,)),
            pltpu.SemaphoreType.DMA((NB,)),
        ],
    )
    def body(a_hbm, b_hbm, src_hbm, dst_hbm, out_hbm,
             si_v, di_v, buf_a, buf_b, sem_i, sem_j, sem_g, sem_o):
        c = lax.axis_index("c")
        s = lax.axis_index("s")
        w = s * NC + c

        cnt = n_chunks // NW + jnp.where(w < (n_chunks % NW), 1, 0)

        def base_of(t):
            return (w + NW * t) * CH

        def start_idx(t, b):
            pltpu.async_copy(src_hbm.at[pl.ds(base_of(t), CH)],
                             si_v.at[b], sem_i.at[b])
            pltpu.async_copy(dst_hbm.at[pl.ds(base_of(t), CH)],
                             di_v.at[b], sem_j.at[b])

        def wait_idx(t, b):
            pltpu.make_async_copy(src_hbm.at[pl.ds(base_of(t), CH)],
                                  si_v.at[b], sem_i.at[b]).wait()
            pltpu.make_async_copy(dst_hbm.at[pl.ds(base_of(t), CH)],
                                  di_v.at[b], sem_j.at[b]).wait()

        def start_gathers(b):
            pltpu.async_copy(a_hbm.at[si_v.at[b]], buf_a.at[b], sem_g.at[b])
            pltpu.async_copy(b_hbm.at[di_v.at[b]], buf_b.at[b], sem_g.at[b])

        def wait_gathers(b):
            pltpu.make_async_copy(a_hbm.at[si_v.at[b]], buf_a.at[b],
                                  sem_g.at[b]).wait()
            pltpu.make_async_copy(b_hbm.at[di_v.at[b]], buf_b.at[b],
                                  sem_g.at[b]).wait()

        def start_out(t, b):
            pltpu.async_copy(buf_a.at[b], out_hbm.at[pl.ds(base_of(t), CH)],
                             sem_o.at[b])

        def wait_out(t, b):
            pltpu.make_async_copy(buf_a.at[b],
                                  out_hbm.at[pl.ds(base_of(t), CH)],
                                  sem_o.at[b]).wait()

        def add_and_out(t, b):
            def add_row(i, cc):
                for j in range(D // 16):
                    sl = pl.ds(16 * j, 16)
                    buf_a[b, i, sl] = buf_a[b, i, sl] + buf_b[b, i, sl]
                return cc
            lax.fori_loop(0, CH, add_row, 0)
            start_out(t, b)

        start_idx(0, 0)

        def step_b(t, b):
            wait_idx(t, b)

            @pl.when(t >= NB)
            def _():
                wait_out(t - NB, b)
            start_gathers(b)

            @pl.when(t + 1 < cnt)
            def _():
                start_idx(t + 1, (b + 1) % NB)

            @pl.when(t >= 1)
            def _():
                wait_gathers((b + NB - 1) % NB)
                add_and_out(t - 1, (b + NB - 1) % NB)

        def step(t, carry):
            for bb in range(NB):
                @pl.when(t % NB == bb)
                def _(bb=bb):
                    step_b(t, bb)
            return carry
        lax.fori_loop(0, cnt, step, 0)

        def tail_b(bl):
            wait_gathers(bl)
            add_and_out(cnt - 1, bl)
            wait_out(cnt - 3, (bl + 1) % NB)
            wait_out(cnt - 2, (bl + 2) % NB)
            wait_out(cnt - 1, bl)

        for bb in range(NB):
            @pl.when((cnt - 1) % NB == bb)
            def _(bb=bb):
                tail_b(bb)

    return body(a_tab, b_tab, src, dst)


def _ln(h, g, b, eps=1e-5):
    m = jnp.mean(h, axis=-1, keepdims=True)
    v = jnp.mean((h - m) ** 2, axis=-1, keepdims=True)
    return (h - m) * lax.rsqrt(v + eps) * g + b


def _tc_node(nodes, msg_parts, nW0x, nW0m, nb0, ng0, nbe0,
             nW1, nb1, ng1, nbe1, eW0a, eW0b):
    N, D = nodes.shape
    BN = 2000
    assert N % BN == 0
    grid = (N // BN,)

    def body(x_ref, m_ref, w0x_ref, w0m_ref, b0_ref, g0_ref, be0_ref,
             w1_ref, b1_ref, g1_ref, be1_ref, wa_ref, wb_ref,
             nn_ref, a_ref, b_ref):
        x = x_ref[...]
        m = m_ref[0] + m_ref[1]
        h = (jnp.dot(x, w0x_ref[...], preferred_element_type=jnp.float32)
             + jnp.dot(m, w0m_ref[...], preferred_element_type=jnp.float32)
             + b0_ref[...])
        h = jnp.maximum(_ln(h, g0_ref[...], be0_ref[...]), 0.0)
        h = jnp.dot(h, w1_ref[...], preferred_element_type=jnp.float32) + b1_ref[...]
        h = jnp.maximum(_ln(h, g1_ref[...], be1_ref[...]), 0.0)
        nn = h + x
        nn_ref[...] = nn
        a_ref[...] = jnp.dot(nn, wa_ref[...], preferred_element_type=jnp.float32)
        b_ref[...] = jnp.dot(nn, wb_ref[...], preferred_element_type=jnp.float32)

    blk = pl.BlockSpec((BN, D), lambda i: (i, 0))
    wspec = pl.BlockSpec((D, D), lambda i: (0, 0))
    vspec = pl.BlockSpec((1, D), lambda i: (0, 0))
    return pl.pallas_call(
        body,
        grid=grid,
        in_specs=[blk, pl.BlockSpec((2, BN, D), lambda i: (0, i, 0)),
                  wspec, wspec, vspec, vspec, vspec,
                  wspec, vspec, vspec, vspec, wspec, wspec],
        out_specs=[blk, blk, blk],
        out_shape=[jax.ShapeDtypeStruct((N, D), jnp.float32)] * 3,
    )(nodes, msg_parts, nW0x, nW0m, nb0, ng0, nbe0,
      nW1, nb1, ng1, nbe1, eW0a, eW0b)


def _tc_edge(edges, gsum, eW0e, eb0, eg0, ebe0, eW1, eb1, eg1, ebe1):
    E, D = edges.shape
    BE = 8000
    assert E % BE == 0
    grid = (E // BE,)

    def body(x_ref, g_ref, w0_ref, b0_ref, g0_ref, be0_ref,
             w1_ref, b1_ref, g1_ref, be1_ref, out_ref):
        x = x_ref[...]
        h = (jnp.dot(x.astype(jnp.bfloat16), w0_ref[...],
                     preferred_element_type=jnp.float32)
             + g_ref[...] + b0_ref[...])
        h = jnp.maximum(_ln(h, g0_ref[...], be0_ref[...]), 0.0)
        h = jnp.dot(h.astype(jnp.bfloat16), w1_ref[...],
                    preferred_element_type=jnp.float32) + b1_ref[...]
        h = jnp.tanh(_ln(h, g1_ref[...], be1_ref[...]))
        out_ref[...] = h + x

    blk = pl.BlockSpec((BE, D), lambda i: (i, 0))
    wspec = pl.BlockSpec((D, D), lambda i: (0, 0))
    vspec = pl.BlockSpec((1, D), lambda i: (0, 0))
    return pl.pallas_call(
        body,
        grid=grid,
        in_specs=[blk, blk, wspec, vspec, vspec, vspec,
                  wspec, vspec, vspec, vspec],
        out_specs=blk,
        out_shape=jax.ShapeDtypeStruct((E, D), jnp.float32),
    )(edges, gsum, eW0e.astype(jnp.bfloat16), eb0, eg0, ebe0,
      eW1.astype(jnp.bfloat16), eb1, eg1, ebe1)


def kernel(nodes, edges, graph,
           eW0, eb0, eg0, ebe0, eW1, eb1, eg1, ebe1,
           nW0, nb0, ng0, nbe0, nW1, nb1, ng1, nbe1):
    N, D = nodes.shape
    E_sz = edges.shape[0]
    src = graph[0]
    dst = graph[1]

    msg_parts = _sc_scatter_add(edges, dst, N)

    r = lambda v: v.reshape(1, D)
    nodes_new, a_tab, b_tab = _tc_node(
        nodes, msg_parts,
        nW0[:D], nW0[D:], r(nb0), r(ng0), r(nbe0),
        nW1, r(nb1), r(ng1), r(nbe1),
        eW0[:D], eW0[D:2 * D])

    gsum = _sc_gather_sum(a_tab, b_tab, src, dst)

    edges_new = _tc_edge(
        edges, gsum, eW0[2 * D:], r(eb0), r(eg0), r(ebe0),
        eW1, r(eb1), r(eg1), r(ebe1))
    return (nodes_new, edges_new)
